# baseline (speedup 1.0000x reference)
import jax
import jax.numpy as jnp
from jax import lax
from jax.experimental import pallas as pl
from jax.experimental.pallas import tpu as pltpu
from jax.experimental.pallas import tpu_sc as plsc

K = 1024
D = 64
NBLK = 512
NW = 32
CHUNK = 128
THR = 2e-3


def _dist_block(x_ref, c_ref, dist_ref, i1_ref, i2_ref, gap_ref):
    x = x_ref[...]
    c = c_ref[...]
    dots = jax.lax.dot_general(
        c, x, (((1,), (1,)), ((), ())),
        preferred_element_type=jnp.float32,
        precision=jax.lax.Precision.HIGHEST)
    cn = jnp.sum(c * c, axis=1, keepdims=True)
    xn = jnp.sum(x * x, axis=1)[None, :]
    dist = cn - 2.0 * dots + xn
    dist_ref[...] = dist

    iota = jax.lax.broadcasted_iota(jnp.int32, (K, NBLK), 0)
    d1 = jnp.min(dist, axis=0)
    i1 = jnp.min(jnp.where(dist == d1[None, :], iota, K), axis=0)
    m2f = (dist <= (d1 + THR)[None, :]).astype(jnp.float32)
    iota2 = jax.lax.broadcasted_iota(jnp.int32, (2, K), 1).astype(jnp.float32)
    rowsel = jax.lax.broadcasted_iota(jnp.int32, (2, K), 0)
    amat = jnp.where(rowsel == 0, iota2, 1.0)
    sums = jax.lax.dot_general(
        amat, m2f, (((1,), (0,)), ((), ())),
        preferred_element_type=jnp.float32,
        precision=jax.lax.Precision.HIGHEST)
    i2 = jnp.clip(sums[0, :] - i1.astype(jnp.float32), 0.0, float(K - 1))
    i1_ref[0, :] = i1
    i2_ref[0, :] = i2.astype(jnp.int32)
    gap_ref[0, :] = sums[1, :]


def _refine_body(x_hbm, c_hbm, i1_hbm, i2_hbm, gap_hbm, out_hbm,
                 i1v, i2v, gv, ov, x16, c1g, c2g, sem):
    ppw = i1v.shape[0]
    nch = ppw // CHUNK
    wid = lax.axis_index("s") * 2 + lax.axis_index("c")
    base = wid * ppw
    cp1 = pltpu.async_copy(i1_hbm.at[0, pl.ds(base, ppw)], i1v, sem)
    cp2 = pltpu.async_copy(i2_hbm.at[0, pl.ds(base, ppw)], i2v, sem)
    cp3 = pltpu.async_copy(gap_hbm.at[0, pl.ds(base, ppw)], gv, sem)
    cp1.wait()
    cp2.wait()
    cp3.wait()

    lanes = lax.iota(jnp.int32, 16)

    def _group(g, carry):
        v1 = i1v[pl.ds(g * 16, 16)]
        near = gv[pl.ds(g * 16, 16)] > 1.5
        cnt = plsc.all_reduce_population_count(near)

        def _heavy():
            v2 = i2v[pl.ds(g * 16, 16)]
            pltpu.sync_copy(x_hbm.at[pl.ds(base + g * 16, 16)], x16)
            pltpu.async_copy(c_hbm.at[v1], c1g, sem).wait()
            pltpu.async_copy(c_hbm.at[v2], c2g, sem).wait()
            e1 = jnp.zeros((16,), jnp.float32)
            e2 = jnp.zeros((16,), jnp.float32)
            for d in range(D):
                col = jnp.full((16,), d, jnp.int32)
                xd = plsc.load_gather(x16, [lanes, col])
                d1 = xd - plsc.load_gather(c1g, [lanes, col])
                d2 = xd - plsc.load_gather(c2g, [lanes, col])
                e1 = e1 + d1 * d1
                e2 = e2 + d2 * d2
            return jnp.where(e1 < e2, v1,
                             jnp.where(e2 < e1, v2, jnp.minimum(v1, v2)))

        sel = lax.cond(cnt[0] > 0, _heavy, lambda: v1)
        ov[pl.ds(g * 16, 16)] = sel
        return carry

    lax.fori_loop(0, ppw // 16, _group, 0)

    pltpu.sync_copy(ov, out_hbm.at[pl.ds(base, ppw)])


def kernel(inputs, centroids):
    n = inputs.shape[0]
    grid = (n // NBLK,)
    dist, i1, i2, gap = pl.pallas_call(
        _dist_block,
        grid=grid,
        in_specs=[
            pl.BlockSpec((NBLK, D), lambda j: (j, 0)),
            pl.BlockSpec((K, D), lambda j: (0, 0)),
        ],
        out_specs=[
            pl.BlockSpec((K, NBLK), lambda j: (0, j)),
            pl.BlockSpec((1, NBLK), lambda j: (0, j)),
            pl.BlockSpec((1, NBLK), lambda j: (0, j)),
            pl.BlockSpec((1, NBLK), lambda j: (0, j)),
        ],
        out_shape=[
            jax.ShapeDtypeStruct((K, n), jnp.float32),
            jax.ShapeDtypeStruct((1, n), jnp.int32),
            jax.ShapeDtypeStruct((1, n), jnp.int32),
            jax.ShapeDtypeStruct((1, n), jnp.float32),
        ],
    )(inputs, centroids)

    ppw = n // NW
    refine = pl.kernel(
        _refine_body,
        out_type=jax.ShapeDtypeStruct((n,), jnp.int32),
        mesh=plsc.VectorSubcoreMesh(core_axis_name="c", subcore_axis_name="s"),
        compiler_params=pltpu.CompilerParams(
            needs_layout_passes=False, use_tc_tiling_on_sc=False),
        scratch_types=[
            pltpu.VMEM((ppw,), jnp.int32),
            pltpu.VMEM((ppw,), jnp.int32),
            pltpu.VMEM((ppw,), jnp.float32),
            pltpu.VMEM((ppw,), jnp.int32),
            pltpu.VMEM((16, D), jnp.float32),
            pltpu.VMEM((16, D), jnp.float32),
            pltpu.VMEM((16, D), jnp.float32),
            pltpu.SemaphoreType.DMA,
        ],
    )
    assign = refine(inputs, centroids, i1, i2, gap)
    return dist, assign


# speedup vs baseline: 1.1088x; 1.1088x over previous
import jax
import jax.numpy as jnp
from jax import lax
from jax.experimental import pallas as pl
from jax.experimental.pallas import tpu as pltpu
from jax.experimental.pallas import tpu_sc as plsc

K = 1024
D = 64
NBLK = 512
NW = 32
CHUNK = 128
GAP_EPS = 0.01


def _dist_block(x_ref, c_ref, dist_ref, i1_ref, i2_ref, gap_ref):
    x = x_ref[...]
    c = c_ref[...]
    dots = jax.lax.dot_general(
        c, x, (((1,), (1,)), ((), ())),
        preferred_element_type=jnp.float32,
        precision=jax.lax.Precision.HIGHEST)
    cn = jnp.sum(c * c, axis=1, keepdims=True)
    xn = jnp.sum(x * x, axis=1)[None, :]
    dist = cn - 2.0 * dots + xn
    dist_ref[...] = dist

    iota = jax.lax.broadcasted_iota(jnp.int32, (K, NBLK), 0)
    d1 = jnp.min(dist, axis=0)
    i1 = jnp.min(jnp.where(dist == d1[None, :], iota, K), axis=0)
    masked = jnp.where(iota == i1[None, :], jnp.inf, dist)
    d2 = jnp.min(masked, axis=0)
    i2 = jnp.min(jnp.where(masked == d2[None, :], iota, K), axis=0)
    i1_ref[0, :] = i1
    i2_ref[0, :] = i2
    gap_ref[0, :] = d2 - d1


def _refine_body(x_hbm, c_hbm, i1_hbm, i2_hbm, gap_hbm, out_hbm,
                 i1v, i2v, gv, ov, x16, c1g, c2g, sem):
    ppw = i1v.shape[0]
    nch = ppw // CHUNK
    wid = lax.axis_index("s") * 2 + lax.axis_index("c")
    base = wid * ppw
    cp1 = pltpu.async_copy(i1_hbm.at[0, pl.ds(base, ppw)], i1v, sem)
    cp2 = pltpu.async_copy(i2_hbm.at[0, pl.ds(base, ppw)], i2v, sem)
    cp3 = pltpu.async_copy(gap_hbm.at[0, pl.ds(base, ppw)], gv, sem)
    cp1.wait()
    cp2.wait()
    cp3.wait()

    lanes = lax.iota(jnp.int32, 16)

    def _group(g, carry):
        v1 = i1v[pl.ds(g * 16, 16)]
        near = gv[pl.ds(g * 16, 16)] < GAP_EPS
        cnt = plsc.all_reduce_population_count(near)

        def _heavy():
            v2 = i2v[pl.ds(g * 16, 16)]
            pltpu.sync_copy(x_hbm.at[pl.ds(base + g * 16, 16)], x16)
            pltpu.async_copy(c_hbm.at[v1], c1g, sem).wait()
            pltpu.async_copy(c_hbm.at[v2], c2g, sem).wait()
            e1 = jnp.zeros((16,), jnp.float32)
            e2 = jnp.zeros((16,), jnp.float32)
            for d in range(D):
                col = jnp.full((16,), d, jnp.int32)
                xd = plsc.load_gather(x16, [lanes, col])
                d1 = xd - plsc.load_gather(c1g, [lanes, col])
                d2 = xd - plsc.load_gather(c2g, [lanes, col])
                e1 = e1 + d1 * d1
                e2 = e2 + d2 * d2
            return jnp.where(e1 < e2, v1,
                             jnp.where(e2 < e1, v2, jnp.minimum(v1, v2)))

        sel = lax.cond(cnt[0] > 0, _heavy, lambda: v1)
        ov[pl.ds(g * 16, 16)] = sel
        return carry

    lax.fori_loop(0, ppw // 16, _group, 0)

    pltpu.sync_copy(ov, out_hbm.at[pl.ds(base, ppw)])


def kernel(inputs, centroids):
    n = inputs.shape[0]
    grid = (n // NBLK,)
    dist, i1, i2, gap = pl.pallas_call(
        _dist_block,
        grid=grid,
        in_specs=[
            pl.BlockSpec((NBLK, D), lambda j: (j, 0)),
            pl.BlockSpec((K, D), lambda j: (0, 0)),
        ],
        out_specs=[
            pl.BlockSpec((K, NBLK), lambda j: (0, j)),
            pl.BlockSpec((1, NBLK), lambda j: (0, j)),
            pl.BlockSpec((1, NBLK), lambda j: (0, j)),
            pl.BlockSpec((1, NBLK), lambda j: (0, j)),
        ],
        out_shape=[
            jax.ShapeDtypeStruct((K, n), jnp.float32),
            jax.ShapeDtypeStruct((1, n), jnp.int32),
            jax.ShapeDtypeStruct((1, n), jnp.int32),
            jax.ShapeDtypeStruct((1, n), jnp.float32),
        ],
    )(inputs, centroids)

    ppw = n // NW
    refine = pl.kernel(
        _refine_body,
        out_type=jax.ShapeDtypeStruct((n,), jnp.int32),
        mesh=plsc.VectorSubcoreMesh(core_axis_name="c", subcore_axis_name="s"),
        compiler_params=pltpu.CompilerParams(
            needs_layout_passes=False, use_tc_tiling_on_sc=False),
        scratch_types=[
            pltpu.VMEM((ppw,), jnp.int32),
            pltpu.VMEM((ppw,), jnp.int32),
            pltpu.VMEM((ppw,), jnp.float32),
            pltpu.VMEM((ppw,), jnp.int32),
            pltpu.VMEM((16, D), jnp.float32),
            pltpu.VMEM((16, D), jnp.float32),
            pltpu.VMEM((16, D), jnp.float32),
            pltpu.SemaphoreType.DMA,
        ],
    )
    assign = refine(inputs, centroids, i1, i2, gap)
    return dist, assign


# speedup vs baseline: 1.1869x; 1.0705x over previous
import jax
import jax.numpy as jnp
from jax import lax
from jax.experimental import pallas as pl
from jax.experimental.pallas import tpu as pltpu
from jax.experimental.pallas import tpu_sc as plsc

K = 1024
D = 64
NBLK = 1024
NW = 32
CHUNK = 128
GAP_EPS = 0.01


def _dist_block(x_ref, c_ref, dist_ref, i1_ref, i2_ref, gap_ref):
    x = x_ref[...]
    c = c_ref[...]
    dots = jax.lax.dot_general(
        c, x, (((1,), (1,)), ((), ())),
        preferred_element_type=jnp.float32,
        precision=jax.lax.Precision.HIGHEST)
    cn = jnp.sum(c * c, axis=1, keepdims=True)
    xn = jnp.sum(x * x, axis=1)[None, :]
    dist = cn - 2.0 * dots + xn
    dist_ref[...] = dist

    iota = jax.lax.broadcasted_iota(jnp.int32, (K, NBLK), 0)
    d1 = jnp.min(dist, axis=0)
    i1 = jnp.min(jnp.where(dist == d1[None, :], iota, K), axis=0)
    masked = jnp.where(iota == i1[None, :], jnp.inf, dist)
    d2 = jnp.min(masked, axis=0)
    i2 = jnp.min(jnp.where(masked == d2[None, :], iota, K), axis=0)
    i1_ref[0, :] = i1
    i2_ref[0, :] = i2
    gap_ref[0, :] = d2 - d1


def _refine_body(x_hbm, c_hbm, i1_hbm, i2_hbm, gap_hbm, out_hbm,
                 i1v, i2v, gv, ov, x16, c1g, c2g, sem):
    ppw = i1v.shape[0]
    nch = ppw // CHUNK
    wid = lax.axis_index("s") * 2 + lax.axis_index("c")
    base = wid * ppw
    cp1 = pltpu.async_copy(i1_hbm.at[0, pl.ds(base, ppw)], i1v, sem)
    cp2 = pltpu.async_copy(i2_hbm.at[0, pl.ds(base, ppw)], i2v, sem)
    cp3 = pltpu.async_copy(gap_hbm.at[0, pl.ds(base, ppw)], gv, sem)
    cp1.wait()
    cp2.wait()
    cp3.wait()

    lanes = lax.iota(jnp.int32, 16)

    def _group(g, carry):
        v1 = i1v[pl.ds(g * 16, 16)]
        near = gv[pl.ds(g * 16, 16)] < GAP_EPS
        cnt = plsc.all_reduce_population_count(near)

        def _heavy():
            v2 = i2v[pl.ds(g * 16, 16)]
            pltpu.sync_copy(x_hbm.at[pl.ds(base + g * 16, 16)], x16)
            pltpu.async_copy(c_hbm.at[v1], c1g, sem).wait()
            pltpu.async_copy(c_hbm.at[v2], c2g, sem).wait()
            e1 = jnp.zeros((16,), jnp.float32)
            e2 = jnp.zeros((16,), jnp.float32)
            for d in range(D):
                col = jnp.full((16,), d, jnp.int32)
                xd = plsc.load_gather(x16, [lanes, col])
                d1 = xd - plsc.load_gather(c1g, [lanes, col])
                d2 = xd - plsc.load_gather(c2g, [lanes, col])
                e1 = e1 + d1 * d1
                e2 = e2 + d2 * d2
            return jnp.where(e1 < e2, v1,
                             jnp.where(e2 < e1, v2, jnp.minimum(v1, v2)))

        sel = lax.cond(cnt[0] > 0, _heavy, lambda: v1)
        ov[pl.ds(g * 16, 16)] = sel
        return carry

    lax.fori_loop(0, ppw // 16, _group, 0)

    pltpu.sync_copy(ov, out_hbm.at[pl.ds(base, ppw)])


def kernel(inputs, centroids):
    n = inputs.shape[0]
    grid = (n // NBLK,)
    dist, i1, i2, gap = pl.pallas_call(
        _dist_block,
        grid=grid,
        in_specs=[
            pl.BlockSpec((NBLK, D), lambda j: (j, 0)),
            pl.BlockSpec((K, D), lambda j: (0, 0)),
        ],
        out_specs=[
            pl.BlockSpec((K, NBLK), lambda j: (0, j)),
            pl.BlockSpec((1, NBLK), lambda j: (0, j)),
            pl.BlockSpec((1, NBLK), lambda j: (0, j)),
            pl.BlockSpec((1, NBLK), lambda j: (0, j)),
        ],
        out_shape=[
            jax.ShapeDtypeStruct((K, n), jnp.float32),
            jax.ShapeDtypeStruct((1, n), jnp.int32),
            jax.ShapeDtypeStruct((1, n), jnp.int32),
            jax.ShapeDtypeStruct((1, n), jnp.float32),
        ],
    )(inputs, centroids)

    ppw = n // NW
    refine = pl.kernel(
        _refine_body,
        out_type=jax.ShapeDtypeStruct((n,), jnp.int32),
        mesh=plsc.VectorSubcoreMesh(core_axis_name="c", subcore_axis_name="s"),
        compiler_params=pltpu.CompilerParams(
            needs_layout_passes=False, use_tc_tiling_on_sc=False),
        scratch_types=[
            pltpu.VMEM((ppw,), jnp.int32),
            pltpu.VMEM((ppw,), jnp.int32),
            pltpu.VMEM((ppw,), jnp.float32),
            pltpu.VMEM((ppw,), jnp.int32),
            pltpu.VMEM((16, D), jnp.float32),
            pltpu.VMEM((16, D), jnp.float32),
            pltpu.VMEM((16, D), jnp.float32),
            pltpu.SemaphoreType.DMA,
        ],
    )
    assign = refine(inputs, centroids, i1, i2, gap)
    return dist, assign
